# baseline (speedup 1.0000x reference)
import functools

import jax
import jax.numpy as jnp
from jax import lax
from jax.experimental import pallas as pl
from jax.experimental.pallas import tpu as pltpu
from jax.experimental.pallas import tpu_sc as plsc

_BM = 256

_CHUNK = 2048


def _argmax_body(a_ref, x_ref, et_ref, c_ref, idx_ref):
    dist = -(a_ref[...]
             - 2.0 * jnp.dot(x_ref[...], et_ref[...],
                             preferred_element_type=jnp.float32)
             + c_ref[...])
    n = dist.shape[-1]
    ids = lax.broadcasted_iota(jnp.int32, dist.shape, 1)
    neg_inf = jnp.float32(-jnp.inf)
    carry_v = None
    carry_i = None
    for lo in range(0, n, _CHUNK):
        hi = min(lo + _CHUNK, n)
        mask = (ids >= lo) & (ids < hi)
        ch = jnp.where(mask, dist, neg_inf)
        m = jnp.max(ch, axis=-1, keepdims=True)
        i = jnp.min(jnp.where(ch == m, ids, jnp.int32(n)), axis=-1,
                    keepdims=True)
        if carry_v is None:
            carry_v, carry_i = m, i
        else:
            upd = m > carry_v
            carry_i = jnp.where(upd, i, carry_i)
            carry_v = jnp.where(upd, m, carry_v)
        carry_v = carry_v.astype(jnp.bfloat16).astype(jnp.float32)
    idx_ref[...] = carry_i[:, 0]


def _nearest_code(a, flat, embed_t, c):
    bt, d = flat.shape
    n = embed_t.shape[1]
    grid = (bt // _BM,)
    return pl.pallas_call(
        _argmax_body,
        grid=grid,
        in_specs=[
            pl.BlockSpec((_BM, 1), lambda i: (i, 0)),
            pl.BlockSpec((_BM, d), lambda i: (i, 0)),
            pl.BlockSpec((d, n), lambda i: (0, 0)),
            pl.BlockSpec((1, n), lambda i: (0, 0)),
        ],
        out_specs=pl.BlockSpec((_BM,), lambda i: (i,)),
        out_shape=jax.ShapeDtypeStruct((bt,), jnp.int32),
    )(a, flat, embed_t, c)


def _make_sc_gather(v, d, b):
    info = plsc.get_sparse_core_info()
    nw = info.num_cores * info.num_subcores
    chunks_per_w = b // (nw * 128)
    mesh = plsc.VectorSubcoreMesh(core_axis_name="c", subcore_axis_name="s")

    @functools.partial(
        pl.kernel,
        mesh=mesh,
        out_type=jax.ShapeDtypeStruct((b // 128, 128, d), jnp.float32),
        scratch_types=[
            pltpu.VMEM((chunks_per_w, 128), jnp.int32),
            pltpu.VMEM((chunks_per_w, 128, d), jnp.float32),
            pltpu.SemaphoreType.DMA,
        ],
    )
    def gather(table_hbm, idx_hbm, out_hbm, idx_v, rows_v, sem):
        wid = lax.axis_index("s") * info.num_cores + lax.axis_index("c")
        base = wid * chunks_per_w
        pltpu.sync_copy(idx_hbm.at[pl.ds(base, chunks_per_w)], idx_v)
        copies = [
            pltpu.async_copy(table_hbm.at[idx_v.at[j]], rows_v.at[j], sem)
            for j in range(chunks_per_w)
        ]
        for cp in copies:
            cp.wait()
        pltpu.sync_copy(rows_v, out_hbm.at[pl.ds(base, chunks_per_w)])

    return gather


def kernel(x, embed):
    shape = x.shape
    d = shape[-1]
    flat = x.reshape(-1, d)
    bt = flat.shape[0]
    embed_t = embed.T
    a = jnp.sum(flat ** 2, axis=1, keepdims=True)
    c = jnp.sum(embed_t ** 2, axis=0, keepdims=True)
    idx_flat = _nearest_code(a, flat, embed_t, c)
    gather = _make_sc_gather(embed.shape[0], d, bt)
    quantize = gather(embed, idx_flat.reshape(bt // 128, 128))
    return (quantize.reshape(shape), idx_flat.reshape(shape[:-1]))


# speedup vs baseline: 1.4511x; 1.4511x over previous
import functools

import jax
import jax.numpy as jnp
from jax import lax
from jax.experimental import pallas as pl
from jax.experimental.pallas import tpu as pltpu
from jax.experimental.pallas import tpu_sc as plsc

_BM = 256

_CHUNK = 2048


def _argmax_body(a_ref, x_ref, et_ref, c_ref, idx_ref):
    a = a_ref[...]
    x = x_ref[...]
    n = et_ref.shape[-1]
    carry_v = None
    carry_i = None
    for lo in range(0, n, _CHUNK):
        hi = min(lo + _CHUNK, n)
        mm = jnp.dot(x, et_ref[:, lo:hi], preferred_element_type=jnp.float32)
        dist = -(a - 2.0 * mm + c_ref[:, lo:hi])
        m = jnp.max(dist, axis=-1, keepdims=True)
        ids = lax.broadcasted_iota(jnp.int32, dist.shape, 1) + jnp.int32(lo)
        i = jnp.min(jnp.where(dist == m, ids, jnp.int32(n)), axis=-1,
                    keepdims=True)
        if carry_v is None:
            carry_v, carry_i = m, i
        else:
            upd = m > carry_v
            carry_i = jnp.where(upd, i, carry_i)
            carry_v = jnp.where(upd, m, carry_v)
        carry_v = carry_v.astype(jnp.bfloat16).astype(jnp.float32)
    idx_ref[...] = carry_i[:, 0]


def _nearest_code(a, flat, embed_t, c):
    bt, d = flat.shape
    n = embed_t.shape[1]
    grid = (bt // _BM,)
    return pl.pallas_call(
        _argmax_body,
        grid=grid,
        in_specs=[
            pl.BlockSpec((_BM, 1), lambda i: (i, 0)),
            pl.BlockSpec((_BM, d), lambda i: (i, 0)),
            pl.BlockSpec((d, n), lambda i: (0, 0)),
            pl.BlockSpec((1, n), lambda i: (0, 0)),
        ],
        out_specs=pl.BlockSpec((_BM,), lambda i: (i,)),
        out_shape=jax.ShapeDtypeStruct((bt,), jnp.int32),
    )(a, flat, embed_t, c)


def _make_sc_gather(v, d, b):
    info = plsc.get_sparse_core_info()
    nw = info.num_cores * info.num_subcores
    chunks_per_w = b // (nw * 128)
    mesh = plsc.VectorSubcoreMesh(core_axis_name="c", subcore_axis_name="s")

    @functools.partial(
        pl.kernel,
        mesh=mesh,
        out_type=jax.ShapeDtypeStruct((b // 128, 128, d), jnp.float32),
        scratch_types=[
            pltpu.VMEM((chunks_per_w, 128), jnp.int32),
            pltpu.VMEM((chunks_per_w, 128, d), jnp.float32),
            pltpu.SemaphoreType.DMA,
        ],
    )
    def gather(table_hbm, idx_hbm, out_hbm, idx_v, rows_v, sem):
        wid = lax.axis_index("s") * info.num_cores + lax.axis_index("c")
        base = wid * chunks_per_w
        pltpu.sync_copy(idx_hbm.at[pl.ds(base, chunks_per_w)], idx_v)
        copies = [
            pltpu.async_copy(table_hbm.at[idx_v.at[j]], rows_v.at[j], sem)
            for j in range(chunks_per_w)
        ]
        for cp in copies:
            cp.wait()
        pltpu.sync_copy(rows_v, out_hbm.at[pl.ds(base, chunks_per_w)])

    return gather


def kernel(x, embed):
    shape = x.shape
    d = shape[-1]
    flat = x.reshape(-1, d)
    bt = flat.shape[0]
    embed_t = embed.T
    a = jnp.sum(flat ** 2, axis=1, keepdims=True)
    c = jnp.sum(embed_t ** 2, axis=0, keepdims=True)
    idx_flat = _nearest_code(a, flat.astype(jnp.bfloat16),
                             embed_t.astype(jnp.bfloat16), c)
    gather = _make_sc_gather(embed.shape[0], d, bt)
    quantize = gather(embed, idx_flat.reshape(bt // 128, 128))
    return (quantize.reshape(shape), idx_flat.reshape(shape[:-1]))


# speedup vs baseline: 1.5143x; 1.0435x over previous
import functools

import jax
import jax.numpy as jnp
from jax import lax
from jax.experimental import pallas as pl
from jax.experimental.pallas import tpu as pltpu
from jax.experimental.pallas import tpu_sc as plsc

_BM = 256
_CHUNK = 2048


def _argmax_body(a_ref, x_ref, e_ref, c_ref, idx_ref):
    a = a_ref[...]
    x = x_ref[...]
    n = e_ref.shape[0]
    nchunks = n // _CHUNK
    dims = (((1,), (1,)), ((), ()))
    ts = []
    carry_v = None
    win_chunk = None
    win_val = None
    for k in range(nchunks):
        lo = k * _CHUNK
        mm = lax.dot_general(x, e_ref[lo:lo + _CHUNK, :], dims,
                             preferred_element_type=jnp.float32)
        t = (a - 2.0 * mm) + c_ref[:, lo:lo + _CHUNK]
        ts.append(t)
        m = jnp.min(t, axis=-1, keepdims=True)
        if carry_v is None:
            carry_v = m
            win_chunk = jnp.zeros_like(m, dtype=jnp.int32)
            win_val = m
        else:
            upd = m < carry_v
            win_chunk = jnp.where(upd, jnp.int32(k), win_chunk)
            win_val = jnp.where(upd, m, win_val)
            carry_v = jnp.where(upd, m, carry_v)
        carry_v = carry_v.astype(jnp.bfloat16).astype(jnp.float32)
    t_all = jnp.concatenate(ts, axis=-1)
    ids = lax.broadcasted_iota(jnp.int32, t_all.shape, 1)
    cid = lax.shift_right_logical(ids, 11)
    hit = (t_all == win_val) & (cid == win_chunk)
    idx_ref[...] = jnp.min(jnp.where(hit, ids, jnp.int32(n)), axis=-1)


def _nearest_code(a, flat, embed, c):
    bt, d = flat.shape
    n = embed.shape[0]
    return pl.pallas_call(
        _argmax_body,
        grid=(bt // _BM,),
        in_specs=[
            pl.BlockSpec((_BM, 1), lambda i: (i, 0)),
            pl.BlockSpec((_BM, d), lambda i: (i, 0)),
            pl.BlockSpec((n, d), lambda i: (0, 0)),
            pl.BlockSpec((1, n), lambda i: (0, 0)),
        ],
        out_specs=pl.BlockSpec((_BM,), lambda i: (i,)),
        out_shape=jax.ShapeDtypeStruct((bt,), jnp.int32),
    )(a, flat, embed, c)


def _make_sc_gather(v, d, b):
    info = plsc.get_sparse_core_info()
    nw = info.num_cores * info.num_subcores
    chunks_per_w = b // (nw * 128)
    mesh = plsc.VectorSubcoreMesh(core_axis_name="c", subcore_axis_name="s")

    @functools.partial(
        pl.kernel,
        mesh=mesh,
        out_type=jax.ShapeDtypeStruct((b // 128, 128, d), jnp.float32),
        scratch_types=[
            pltpu.VMEM((chunks_per_w, 128), jnp.int32),
            pltpu.VMEM((chunks_per_w, 128, d), jnp.float32),
            pltpu.SemaphoreType.DMA,
        ],
    )
    def gather(table_hbm, idx_hbm, out_hbm, idx_v, rows_v, sem):
        wid = lax.axis_index("s") * info.num_cores + lax.axis_index("c")
        base = wid * chunks_per_w
        pltpu.sync_copy(idx_hbm.at[pl.ds(base, chunks_per_w)], idx_v)
        copies = [
            pltpu.async_copy(table_hbm.at[idx_v.at[j]], rows_v.at[j], sem)
            for j in range(chunks_per_w)
        ]
        for cp in copies:
            cp.wait()
        pltpu.sync_copy(rows_v, out_hbm.at[pl.ds(base, chunks_per_w)])

    return gather


def kernel(x, embed):
    shape = x.shape
    d = shape[-1]
    flat = x.reshape(-1, d)
    bt = flat.shape[0]
    a = jnp.sum(flat ** 2, axis=1, keepdims=True)
    c = jnp.sum(embed.T ** 2, axis=0, keepdims=True)
    idx_flat = _nearest_code(a, flat.astype(jnp.bfloat16),
                             embed.astype(jnp.bfloat16), c)
    gather = _make_sc_gather(embed.shape[0], d, bt)
    quantize = gather(embed, idx_flat.reshape(bt // 128, 128))
    return (quantize.reshape(shape), idx_flat.reshape(shape[:-1]))


# speedup vs baseline: 1.7407x; 1.1495x over previous
import functools

import jax
import jax.numpy as jnp
from jax import lax
from jax.experimental import pallas as pl
from jax.experimental.pallas import tpu as pltpu
from jax.experimental.pallas import tpu_sc as plsc

_BM = 256
_CHUNK = 2048


def _argmax_body(a_ref, x_ref, e_ref, c_ref, idx_ref, eb_ref):
    @pl.when(pl.program_id(0) == 0)
    def _cast_codebook():
        eb_ref[...] = e_ref[...].astype(jnp.bfloat16)

    a = a_ref[...]
    x = x_ref[...].astype(jnp.bfloat16)
    n = e_ref.shape[0]
    dims = (((1,), (1,)), ((), ()))
    carry_v = None
    carry_i = None
    for k in range(n // _CHUNK):
        lo = k * _CHUNK
        mm = lax.dot_general(x, eb_ref[lo:lo + _CHUNK, :], dims,
                             preferred_element_type=jnp.float32)
        t = (a - 2.0 * mm) + c_ref[:, lo:lo + _CHUNK]
        m = jnp.min(t, axis=-1, keepdims=True)
        ids = lax.broadcasted_iota(jnp.int32, t.shape, 1) + jnp.int32(lo)
        i = jnp.min(jnp.where(t == m, ids, jnp.int32(n)), axis=-1,
                    keepdims=True)
        if carry_v is None:
            carry_v, carry_i = m, i
        else:
            upd = m < carry_v
            carry_i = jnp.where(upd, i, carry_i)
            carry_v = jnp.where(upd, m, carry_v)
        carry_v = carry_v.astype(jnp.bfloat16).astype(jnp.float32)
    idx_ref[...] = carry_i[:, 0]


def _nearest_code(a, flat, embed, c):
    bt, d = flat.shape
    n = embed.shape[0]
    return pl.pallas_call(
        _argmax_body,
        grid=(bt // _BM,),
        in_specs=[
            pl.BlockSpec((_BM, 1), lambda i: (i, 0)),
            pl.BlockSpec((_BM, d), lambda i: (i, 0)),
            pl.BlockSpec((n, d), lambda i: (0, 0)),
            pl.BlockSpec((1, n), lambda i: (0, 0)),
        ],
        out_specs=pl.BlockSpec((_BM,), lambda i: (i,)),
        out_shape=jax.ShapeDtypeStruct((bt,), jnp.int32),
        scratch_shapes=[pltpu.VMEM((n, d), jnp.bfloat16)],
    )(a, flat, embed, c)


def _make_sc_gather(v, d, b):
    info = plsc.get_sparse_core_info()
    nw = info.num_cores * info.num_subcores
    chunks_per_w = b // (nw * 128)
    mesh = plsc.VectorSubcoreMesh(core_axis_name="c", subcore_axis_name="s")

    @functools.partial(
        pl.kernel,
        mesh=mesh,
        out_type=jax.ShapeDtypeStruct((b // 128, 128, d), jnp.float32),
        scratch_types=[
            pltpu.VMEM((chunks_per_w, 128), jnp.int32),
            pltpu.VMEM((chunks_per_w, 128, d), jnp.float32),
            pltpu.SemaphoreType.DMA,
        ],
    )
    def gather(table_hbm, idx_hbm, out_hbm, idx_v, rows_v, sem):
        wid = lax.axis_index("s") * info.num_cores + lax.axis_index("c")
        base = wid * chunks_per_w
        pltpu.sync_copy(idx_hbm.at[pl.ds(base, chunks_per_w)], idx_v)
        copies = [
            pltpu.async_copy(table_hbm.at[idx_v.at[j]], rows_v.at[j], sem)
            for j in range(chunks_per_w)
        ]
        for cp in copies:
            cp.wait()
        pltpu.sync_copy(rows_v, out_hbm.at[pl.ds(base, chunks_per_w)])

    return gather


def kernel(x, embed):
    shape = x.shape
    d = shape[-1]
    flat = x.reshape(-1, d)
    bt = flat.shape[0]
    a = jnp.sum(flat ** 2, axis=1, keepdims=True)
    c = jnp.sum(embed.T ** 2, axis=0, keepdims=True)
    idx_flat = _nearest_code(a, flat, embed, c)
    gather = _make_sc_gather(embed.shape[0], d, bt)
    quantize = gather(embed, idx_flat.reshape(bt // 128, 128))
    return (quantize.reshape(shape), idx_flat.reshape(shape[:-1]))
